# baseline (speedup 1.0000x reference)
import functools

import jax
import jax.numpy as jnp
from jax import lax
from jax.experimental import pallas as pl
from jax.experimental.pallas import tpu as pltpu
from jax.experimental.pallas import tpu_sc as plsc

K = 64
HIST = 200
NW = 16
RPW = 32
VALID = 25


def _din_body(idx_hbm, uid_hbm, mid_hbm, ut_hbm, mt_hbm, ct_hbm,
              w0_hbm, b0_hbm, w1_hbm, b1_hbm, w2_hbm, b2_hbm,
              out_hbm,
              idx_v, rows_v, stage_v, shared, part_v,
              uid_v, mid_v, a_v, b_v,
              w0_v, b0_v, w1_v, b1_v, w2_v, b2_v,
              fea_v, h_v, g_v, res_v,
              gsem, asem, bsem):
    cid = lax.axis_index("c")
    sid = lax.axis_index("s")

    @pl.when(cid == 0)
    def _core0():
        pltpu.sync_copy(idx_hbm.at[sid], idx_v)
        gather = pltpu.async_copy(ct_hbm.at[idx_v], rows_v, gsem)

        @pl.when(sid == 0)
        def _prefetch_w0():
            pltpu.sync_copy(uid_hbm, uid_v)
            pltpu.sync_copy(mid_hbm, mid_v)
            a_cp = pltpu.async_copy(ut_hbm.at[uid_v], a_v, asem)
            b_cp = pltpu.async_copy(mt_hbm.at[mid_v], b_v, bsem)
            pltpu.sync_copy(w0_hbm, w0_v)
            pltpu.sync_copy(b0_hbm, b0_v)
            pltpu.sync_copy(w1_hbm, w1_v)
            pltpu.sync_copy(b1_hbm, b1_v)
            pltpu.sync_copy(w2_hbm, w2_v)
            pltpu.sync_copy(b2_hbm, b2_v)
            a_cp.wait()
            b_cp.wait()

        gather.wait()
        acc = [rows_v[0, pl.ds(16 * i, 16)] for i in range(4)]
        for j in range(1, VALID):
            for i in range(4):
                acc[i] = acc[i] + rows_v[j, pl.ds(16 * i, 16)]
        for i in range(4):
            stage_v[pl.ds(16 * i, 16)] = acc[i]
        pltpu.sync_copy(stage_v, shared.at[sid])
        plsc.subcore_barrier()

        @pl.when(sid == 0)
        def _finish():
            pltpu.sync_copy(shared, part_v)
            inv = 1.0 / float(HIST)
            for i in range(4):
                fea_v[pl.ds(i * 16, 16)] = a_v[0, pl.ds(16 * i, 16)]
                fea_v[pl.ds(64 + i * 16, 16)] = b_v[0, pl.ds(16 * i, 16)]
                csum = part_v[0, pl.ds(16 * i, 16)]
                dsum = part_v[8, pl.ds(16 * i, 16)]
                for j in range(1, 8):
                    csum = csum + part_v[j, pl.ds(16 * i, 16)]
                    dsum = dsum + part_v[8 + j, pl.ds(16 * i, 16)]
                fea_v[pl.ds(128 + i * 16, 16)] = csum * inv
                fea_v[pl.ds(192 + i * 16, 16)] = dsum * inv

            h0 = b0_v[pl.ds(0, 16)]
            h1 = b0_v[pl.ds(16, 16)]
            for t in range(16):
                chunk = fea_v[pl.ds(16 * t, 16)]
                for l in range(16):
                    f = chunk[l]
                    k = 16 * t + l
                    h0 = h0 + f * w0_v[k, pl.ds(0, 16)]
                    h1 = h1 + f * w0_v[k, pl.ds(16, 16)]
            h0 = jnp.maximum(h0, 0.0)
            h1 = jnp.maximum(h1, 0.0)

            g = b1_v[...]
            for k in range(16):
                g = g + h0[k] * w1_v[k]
            for k in range(4):
                g = g + h1[k] * w1_v[16 + k]
            g = jnp.maximum(g, 0.0)

            r = b2_v[...]
            for k in range(8):
                r = r + g[k] * w2_v[k]
            delta = r[0] - r[1]
            dvec = jnp.zeros((16,), jnp.float32) + delta
            res_v[...] = 1.0 / (1.0 + jnp.exp(dvec))
            pltpu.sync_copy(res_v, out_hbm)


@jax.jit
def _din_sc(idx2d, uid8, mid8, user_table, movie_table, movie_cate_table,
            w0p, b0p, w1p, b1p, w2p, b2p):
    mesh = plsc.VectorSubcoreMesh(core_axis_name="c", subcore_axis_name="s")
    f32 = jnp.float32
    run = functools.partial(
        pl.kernel,
        out_type=jax.ShapeDtypeStruct((16,), f32),
        mesh=mesh,
        scratch_types=[
            pltpu.VMEM((RPW,), jnp.int32),
            pltpu.VMEM((RPW, K), f32),
            pltpu.VMEM((K,), f32),
            pltpu.VMEM_SHARED((NW, K), f32),
            pltpu.VMEM((NW, K), f32),
            pltpu.VMEM((8,), jnp.int32),
            pltpu.VMEM((8,), jnp.int32),
            pltpu.VMEM((8, K), f32),
            pltpu.VMEM((8, K), f32),
            pltpu.VMEM((256, 32), f32),
            pltpu.VMEM((32,), f32),
            pltpu.VMEM((20, 16), f32),
            pltpu.VMEM((16,), f32),
            pltpu.VMEM((8, 16), f32),
            pltpu.VMEM((16,), f32),
            pltpu.VMEM((256,), f32),
            pltpu.VMEM((32,), f32),
            pltpu.VMEM((16,), f32),
            pltpu.VMEM((16,), f32),
            pltpu.SemaphoreType.DMA,
            pltpu.SemaphoreType.DMA,
            pltpu.SemaphoreType.DMA,
        ],
        compiler_params=pltpu.CompilerParams(use_tc_tiling_on_sc=False),
    )(_din_body)
    return run(idx2d, uid8, mid8, user_table, movie_table, movie_cate_table,
               w0p, b0p, w1p, b1p, w2p, b2p)


def kernel(user_id, movie_id, movie_cate, user_rate, user_table, movie_table,
           movie_cate_table, W0, b0, W1, b1, W2, b2):
    i32 = jnp.int32
    cat = jnp.concatenate([movie_cate, user_rate]).astype(i32).reshape(NW, VALID)
    idx2d = jnp.pad(cat, ((0, 0), (0, RPW - VALID)))
    uid8 = jnp.broadcast_to(user_id.astype(i32), (8,))
    mid8 = jnp.broadcast_to(movie_id.astype(i32), (8,))
    w0p = jnp.pad(W0, ((0, 0), (0, 12)))
    b0p = jnp.pad(b0, (0, 12))
    w1p = jnp.pad(W1, ((0, 0), (0, 8)))
    b1p = jnp.pad(b1, (0, 8))
    w2p = jnp.pad(W2, ((0, 0), (0, 14)))
    b2p = jnp.pad(b2, (0, 14))
    out = _din_sc(idx2d, uid8, mid8, user_table, movie_table, movie_cate_table,
                  w0p, b0p, w1p, b1p, w2p, b2p)
    return out[1:2]


# speedup vs baseline: 1.3341x; 1.3341x over previous
import functools

import jax
import jax.numpy as jnp
from jax import lax
from jax.experimental import pallas as pl
from jax.experimental.pallas import tpu as pltpu
from jax.experimental.pallas import tpu_sc as plsc

K = 64
HIST = 200
NW = 16
VALID = 25


def _din_body(idx_hbm, ut_hbm, mt_hbm, ct_hbm,
              w0_hbm, b0_hbm, w1_hbm, b1_hbm, w2_hbm, b2_hbm,
              out_hbm,
              idx_v, rows_v, ab_v, stage_v, shared, part_v,
              w0_v, b0_v, w1_v, b1_v, w2_v, b2_v,
              res_v, gsem, absem):
    cid = lax.axis_index("c")
    sid = lax.axis_index("s")

    @pl.when(cid == 0)
    def _core0():
        pltpu.sync_copy(idx_hbm.at[sid], idx_v)
        i0 = idx_v[pl.ds(0, 16)]
        i1 = idx_v[pl.ds(16, 16)]
        idxs = [i0[l] for l in range(16)] + [i1[l] for l in range(VALID - 16)]
        copies = [
            pltpu.async_copy(ct_hbm.at[pl.ds(idxs[j], 1)],
                             rows_v.at[pl.ds(j, 1)], gsem)
            for j in range(VALID)
        ]

        @pl.when(sid == 0)
        def _prefetch():
            i2 = idx_v[pl.ds(32, 16)]
            a_cp = pltpu.async_copy(ut_hbm.at[pl.ds(i2[0], 1)],
                                    ab_v.at[pl.ds(0, 1)], absem)
            b_cp = pltpu.async_copy(mt_hbm.at[pl.ds(i2[1], 1)],
                                    ab_v.at[pl.ds(1, 1)], absem)
            pltpu.sync_copy(w0_hbm, w0_v)
            pltpu.sync_copy(b0_hbm, b0_v)
            pltpu.sync_copy(w1_hbm, w1_v)
            pltpu.sync_copy(b1_hbm, b1_v)
            pltpu.sync_copy(w2_hbm, w2_v)
            pltpu.sync_copy(b2_hbm, b2_v)
            a_cp.wait()
            b_cp.wait()

        for c in copies:
            c.wait()
        acc = [rows_v[0, pl.ds(16 * i, 16)] for i in range(4)]
        for j in range(1, VALID):
            for i in range(4):
                acc[i] = acc[i] + rows_v[j, pl.ds(16 * i, 16)]
        for i in range(4):
            stage_v[pl.ds(16 * i, 16)] = acc[i]
        pltpu.sync_copy(stage_v, shared.at[sid])
        plsc.subcore_barrier()

        @pl.when(sid == 0)
        def _finish():
            pltpu.sync_copy(shared, part_v)
            inv = 1.0 / float(HIST)
            fea = []
            for i in range(4):
                fea.append(ab_v[0, pl.ds(16 * i, 16)])
            for i in range(4):
                fea.append(ab_v[1, pl.ds(16 * i, 16)])
            for seg in range(2):
                for i in range(4):
                    s = part_v[8 * seg, pl.ds(16 * i, 16)]
                    for j in range(1, 8):
                        s = s + part_v[8 * seg + j, pl.ds(16 * i, 16)]
                    fea.append(s * inv)

            h0 = b0_v[pl.ds(0, 16)]
            h1 = b0_v[pl.ds(16, 16)]
            for t in range(16):
                chunk = fea[t]
                for l in range(16):
                    f = chunk[l]
                    k = 16 * t + l
                    h0 = h0 + f * w0_v[k, pl.ds(0, 16)]
                    h1 = h1 + f * w0_v[k, pl.ds(16, 16)]
            h0 = jnp.maximum(h0, 0.0)
            h1 = jnp.maximum(h1, 0.0)

            g = b1_v[pl.ds(0, 16)]
            for k in range(16):
                g = g + h0[k] * w1_v[k, pl.ds(0, 16)]
            for k in range(4):
                g = g + h1[k] * w1_v[16 + k, pl.ds(0, 16)]
            g = jnp.maximum(g, 0.0)

            r = b2_v[pl.ds(0, 16)]
            for k in range(8):
                r = r + g[k] * w2_v[k, pl.ds(0, 16)]
            delta = r[0] - r[1]
            dvec = jnp.zeros((16,), jnp.float32) + delta
            res_v[...] = 1.0 / (1.0 + jnp.exp(dvec))
            pltpu.sync_copy(res_v, out_hbm)


@jax.jit
def _din_sc(idx2d, user_table, movie_table, movie_cate_table,
            w0p, b0p, w1p, b1p, w2p, b2p):
    mesh = plsc.VectorSubcoreMesh(core_axis_name="c", subcore_axis_name="s")
    f32 = jnp.float32
    run = functools.partial(
        pl.kernel,
        out_type=jax.ShapeDtypeStruct((16,), f32),
        mesh=mesh,
        scratch_types=[
            pltpu.VMEM((128,), jnp.int32),
            pltpu.VMEM((VALID, K), f32),
            pltpu.VMEM((2, K), f32),
            pltpu.VMEM((K,), f32),
            pltpu.VMEM_SHARED((NW, K), f32),
            pltpu.VMEM((NW, K), f32),
            pltpu.VMEM((256, 32), f32),
            pltpu.VMEM((32,), f32),
            pltpu.VMEM((20, 16), f32),
            pltpu.VMEM((16,), f32),
            pltpu.VMEM((8, 16), f32),
            pltpu.VMEM((16,), f32),
            pltpu.VMEM((16,), f32),
            pltpu.SemaphoreType.DMA,
            pltpu.SemaphoreType.DMA,
        ],
    )(_din_body)
    return run(idx2d, user_table, movie_table, movie_cate_table,
               w0p, b0p, w1p, b1p, w2p, b2p)


def kernel(user_id, movie_id, movie_cate, user_rate, user_table, movie_table,
           movie_cate_table, W0, b0, W1, b1, W2, b2):
    i32 = jnp.int32
    cat = jnp.concatenate([movie_cate, user_rate]).astype(i32).reshape(NW, VALID)
    idx2d = jnp.pad(cat, ((0, 0), (0, 128 - VALID)))
    um = jnp.concatenate([user_id.astype(i32), movie_id.astype(i32)])
    idx2d = lax.dynamic_update_slice(idx2d, um[None, :], (0, 32))
    w0p = jnp.pad(W0, ((0, 0), (0, 12)))
    b0p = jnp.pad(b0, (0, 12))
    w1p = jnp.pad(W1, ((0, 0), (0, 8)))
    b1p = jnp.pad(b1, (0, 8))
    w2p = jnp.pad(W2, ((0, 0), (0, 14)))
    b2p = jnp.pad(b2, (0, 14))
    out = _din_sc(idx2d, user_table, movie_table, movie_cate_table,
                  w0p, b0p, w1p, b1p, w2p, b2p)
    return out[1:2]
